# baseline (speedup 1.0000x reference)
import functools

import jax
import jax.numpy as jnp
from jax import lax
from jax.experimental import pallas as pl
from jax.experimental.pallas import tpu as pltpu
from jax.experimental.pallas import tpu_sc as plsc

_N, _E, _IN, _H, _OUT, _ED, _G = 10000, 40000, 64, 32, 10, 16, 500

_NW = 32
_EP = 40960
_EPW = 1280
_ECH = 128
_EK = _EPW // _ECH
_NPAD = 10016
_NP = 10240
_NPW = _NP // _NW
_NCH = 64
_NK = _NPW // _NCH
_GP = 512
_HX = _H + 16


def _sc_mesh():
    return plsc.VectorSubcoreMesh(
        core_axis_name="c", subcore_axis_name="s", num_cores=2, num_subcores=16
    )


_SC_PARAMS = pltpu.CompilerParams(use_tc_tiling_on_sc=False)


def _make_gather(d):
    @functools.partial(
        pl.kernel,
        out_type=jax.ShapeDtypeStruct((_EP, d), jnp.float32),
        mesh=_sc_mesh(),
        compiler_params=_SC_PARAMS,
        scratch_types=[
            pltpu.VMEM((_EK, _ECH), jnp.int32),
            pltpu.VMEM((_EPW, d), jnp.float32),
            pltpu.SemaphoreType.DMA,
        ],
    )
    def gk(table, idx, out, idx_v, rows_v, sem):
        wid = lax.axis_index("s") * 2 + lax.axis_index("c")
        pltpu.sync_copy(idx.at[wid], idx_v)
        cps = [
            pltpu.async_copy(
                table.at[idx_v.at[j]], rows_v.at[pl.ds(j * _ECH, _ECH)], sem
            )
            for j in range(_EK)
        ]
        for cp in cps:
            cp.wait()
        pltpu.sync_copy(rows_v, out.at[pl.ds(wid * _EPW, _EPW)])

    return gk


def _make_scatter():
    rpt = _NPAD // 16

    @functools.partial(
        pl.kernel,
        out_type=jax.ShapeDtypeStruct((2, _NPAD, _H), jnp.float32),
        mesh=_sc_mesh(),
        compiler_params=_SC_PARAMS,
        scratch_types=[
            pltpu.VMEM((_EK, _ECH), jnp.int32),
            pltpu.VMEM((_EPW, _H), jnp.float32),
            pltpu.VMEM_SHARED((_NPAD, _H), jnp.float32),
            pltpu.SemaphoreType.DMA,
        ],
    )
    def sk(msg, idx, zeros, out, idx_v, rows_v, accum, sem):
        c = lax.axis_index("c")
        s = lax.axis_index("s")
        wid = s * 2 + c
        pltpu.sync_copy(zeros.at[pl.ds(s * rpt, rpt)], accum.at[pl.ds(s * rpt, rpt)])
        pltpu.sync_copy(idx.at[wid], idx_v)
        pltpu.sync_copy(msg.at[pl.ds(wid * _EPW, _EPW)], rows_v)
        plsc.subcore_barrier()
        for j in range(_EK):
            pltpu.sync_copy(
                rows_v.at[pl.ds(j * _ECH, _ECH)], accum.at[idx_v.at[j]], add=True
            )
        plsc.subcore_barrier()
        pltpu.sync_copy(accum.at[pl.ds(s * rpt, rpt)], out.at[c, pl.ds(s * rpt, rpt)])

    return sk


def _make_pool():
    rpt = _GP // 16

    @functools.partial(
        pl.kernel,
        out_type=jax.ShapeDtypeStruct((2, _GP, _HX), jnp.float32),
        mesh=_sc_mesh(),
        compiler_params=_SC_PARAMS,
        scratch_types=[
            pltpu.VMEM((_NK, _NCH), jnp.int32),
            pltpu.VMEM((_NPW, _HX), jnp.float32),
            pltpu.VMEM_SHARED((_GP, _HX), jnp.float32),
            pltpu.SemaphoreType.DMA,
        ],
    )
    def pk(hx, idx, zeros, out, idx_v, rows_v, accum, sem):
        c = lax.axis_index("c")
        s = lax.axis_index("s")
        wid = s * 2 + c
        pltpu.sync_copy(zeros.at[pl.ds(s * rpt, rpt)], accum.at[pl.ds(s * rpt, rpt)])
        pltpu.sync_copy(idx.at[wid], idx_v)
        pltpu.sync_copy(hx.at[pl.ds(wid * _NPW, _NPW)], rows_v)
        plsc.subcore_barrier()
        for j in range(_NK):
            pltpu.sync_copy(
                rows_v.at[pl.ds(j * _NCH, _NCH)], accum.at[idx_v.at[j]], add=True
            )
        plsc.subcore_barrier()
        pltpu.sync_copy(accum.at[pl.ds(s * rpt, rpt)], out.at[c, pl.ds(s * rpt, rpt)])

    return pk


def _edge_call(xs, ea, w2, bm, ic):
    te = 512
    grid = _EP // te
    per = 128 // ic
    nb = _ED // per

    def body(xs_ref, ea_ref, w_ref, b_ref, o_ref):
        xv = xs_ref[...]
        ev = ea_ref[...]
        blocks = []
        for v in range(nb):
            pieces = [
                ev[:, v * per + p : v * per + p + 1] * xv for p in range(per)
            ]
            blocks.append(jnp.concatenate(pieces, axis=1))
        xa = jnp.concatenate(blocks, axis=1)
        o_ref[...] = jnp.dot(
            xv, b_ref[...], preferred_element_type=jnp.float32
        ) + jnp.dot(xa, w_ref[...], preferred_element_type=jnp.float32)

    return pl.pallas_call(
        body,
        grid=(grid,),
        in_specs=[
            pl.BlockSpec((te, ic), lambda i: (i, 0)),
            pl.BlockSpec((te, _ED), lambda i: (i, 0)),
            pl.BlockSpec((_ED * ic, _H), lambda i: (0, 0)),
            pl.BlockSpec((ic, _H), lambda i: (0, 0)),
        ],
        out_specs=pl.BlockSpec((te, _H), lambda i: (i, 0)),
        out_shape=jax.ShapeDtypeStruct((_EP, _H), jnp.float32),
    )(xs, ea, w2, bm)


def _node_call(p, xin, root, bias, ic):
    tn = 2000
    grid = _N // tn

    def body(p_ref, x_ref, r_ref, b_ref, o_ref):
        acc = (
            p_ref[0]
            + p_ref[1]
            + jnp.dot(x_ref[...], r_ref[...], preferred_element_type=jnp.float32)
            + b_ref[...]
        )
        o_ref[...] = jnp.maximum(acc, 0.0)

    return pl.pallas_call(
        body,
        grid=(grid,),
        in_specs=[
            pl.BlockSpec((2, tn, _H), lambda i: (0, i, 0)),
            pl.BlockSpec((tn, ic), lambda i: (i, 0)),
            pl.BlockSpec((ic, _H), lambda i: (0, 0)),
            pl.BlockSpec((1, _H), lambda i: (0, 0)),
        ],
        out_specs=pl.BlockSpec((tn, _H), lambda i: (i, 0)),
        out_shape=jax.ShapeDtypeStruct((_N, _H), jnp.float32),
    )(p, xin, root, bias.reshape(1, _H))


def _node3_call(p, xin, root, bias):
    tn = 2000
    grid = _N // tn

    def body(p_ref, x_ref, r_ref, b_ref, emb_ref, hx_ref):
        emb = (
            p_ref[0]
            + p_ref[1]
            + jnp.dot(x_ref[...], r_ref[...], preferred_element_type=jnp.float32)
            + b_ref[...]
        )
        emb_ref[...] = emb
        hr = jnp.maximum(emb, 0.0)
        hx_ref[...] = jnp.concatenate(
            [hr, jnp.ones((tn, _HX - _H), jnp.float32)], axis=1
        )

    return pl.pallas_call(
        body,
        grid=(grid,),
        in_specs=[
            pl.BlockSpec((2, tn, _H), lambda i: (0, i, 0)),
            pl.BlockSpec((tn, _H), lambda i: (i, 0)),
            pl.BlockSpec((_H, _H), lambda i: (0, 0)),
            pl.BlockSpec((1, _H), lambda i: (0, 0)),
        ],
        out_specs=[
            pl.BlockSpec((tn, _H), lambda i: (i, 0)),
            pl.BlockSpec((tn, _HX), lambda i: (i, 0)),
        ],
        out_shape=[
            jax.ShapeDtypeStruct((_N, _H), jnp.float32),
            jax.ShapeDtypeStruct((_N, _HX), jnp.float32),
        ],
    )(p, xin, root, bias.reshape(1, _H))


def _head_call(pp, P1, pb1, P2, pb2):
    def body(p_ref, w1_ref, b1_ref, w2_ref, b2_ref, o_ref):
        s = p_ref[0] + p_ref[1]
        tot = s[:, : _H]
        cnt = s[:, _H : _H + 1]
        pooled = tot / jnp.maximum(cnt, 1.0)
        t = (
            jnp.dot(pooled, w1_ref[...], preferred_element_type=jnp.float32)
            + b1_ref[...]
        )
        o_ref[...] = (
            jnp.dot(t, w2_ref[...], preferred_element_type=jnp.float32) + b2_ref[...]
        )

    return pl.pallas_call(
        body,
        grid=(1,),
        in_specs=[
            pl.BlockSpec((2, _GP, _HX), lambda i: (0, 0, 0)),
            pl.BlockSpec((_H, _H), lambda i: (0, 0)),
            pl.BlockSpec((1, _H), lambda i: (0, 0)),
            pl.BlockSpec((_H, _OUT), lambda i: (0, 0)),
            pl.BlockSpec((1, _OUT), lambda i: (0, 0)),
        ],
        out_specs=pl.BlockSpec((_GP, _OUT), lambda i: (0, 0)),
        out_shape=jax.ShapeDtypeStruct((_GP, _OUT), jnp.float32),
    )(pp, P1, pb1.reshape(1, _H), P2, pb2.reshape(1, _OUT))


def kernel(x, edge_index, edge_attr, batch,
           We1, be1, root1, bias1,
           We2, be2, root2, bias2,
           root3, bias3,
           P1, pb1, P2, pb2):
    f32 = jnp.float32
    src = edge_index[0].astype(jnp.int32)
    dst = edge_index[1].astype(jnp.int32)
    srcp = jnp.concatenate([src, jnp.zeros((_EP - _E,), jnp.int32)]).reshape(
        _NW, _EK, _ECH
    )
    dstp = jnp.concatenate([dst, jnp.full((_EP - _E,), _N, jnp.int32)]).reshape(
        _NW, _EK, _ECH
    )
    bat = jnp.concatenate(
        [batch.astype(jnp.int32), jnp.full((_NP - _N,), _G, jnp.int32)]
    ).reshape(_NW, _NK, _NCH)
    eap = jnp.concatenate([edge_attr, jnp.zeros((_EP - _E, _ED), f32)])
    zn = jnp.zeros((_NPAD, _H), f32)
    zg = jnp.zeros((_GP, _HX), f32)
    wr1 = We1.reshape(_ED * _IN, _H)
    bm1 = be1.reshape(_IN, _H)
    wr2 = We2.reshape(_ED * _H, _H)
    bm2 = be2.reshape(_H, _H)

    g64 = _make_gather(_IN)
    g32 = _make_gather(_H)
    sk = _make_scatter()
    pk = _make_pool()

    xs1 = g64(x, srcp)
    m1 = _edge_call(xs1, eap, wr1, bm1, _IN)
    p1 = sk(m1, dstp, zn)
    h1 = _node_call(p1, x, root1, bias1, _IN)

    xs2 = g32(h1, srcp)
    m2 = _edge_call(xs2, eap, wr2, bm2, _H)
    p2 = sk(m2, dstp, zn)
    h2 = _node_call(p2, h1, root2, bias2, _H)

    xs3 = g32(h2, srcp)
    m3 = _edge_call(xs3, eap, wr2, bm2, _H)
    p3 = sk(m3, dstp, zn)
    emb, hext = _node3_call(p3, h2, root3, bias3)

    hxp = jnp.concatenate([hext, jnp.zeros((_NP - _N, _HX), f32)])
    pp = pk(hxp, bat, zg)
    out = _head_call(pp, P1, pb1, P2, pb2)[: _G]
    return (emb, out)


# speedup vs baseline: 1.6798x; 1.6798x over previous
import functools

import jax
import jax.numpy as jnp
from jax import lax
from jax.experimental import pallas as pl
from jax.experimental.pallas import tpu as pltpu
from jax.experimental.pallas import tpu_sc as plsc

_N, _E, _IN, _H, _OUT, _ED, _G = 10000, 40000, 64, 32, 10, 16, 500

_NW = 32
_EP = 40960
_EPW = 1280
_ECH = 128
_EK = _EPW // _ECH
_NPAD = 10016
_NP = 10240
_NPW = _NP // _NW
_NCH = 64
_NK = _NPW // _NCH
_GP = 512
_HX = _H + 16


def _sc_mesh():
    return plsc.VectorSubcoreMesh(
        core_axis_name="c", subcore_axis_name="s", num_cores=2, num_subcores=16
    )


_SC_PARAMS = pltpu.CompilerParams(use_tc_tiling_on_sc=False)


def _make_gather(d):
    @functools.partial(
        pl.kernel,
        out_type=jax.ShapeDtypeStruct((_EP, d), jnp.float32),
        mesh=_sc_mesh(),
        compiler_params=_SC_PARAMS,
        scratch_types=[
            pltpu.VMEM((_EK, _ECH), jnp.int32),
            pltpu.VMEM((_EPW, d), jnp.float32),
            pltpu.SemaphoreType.DMA,
        ],
    )
    def gk(table, idx, out, idx_v, rows_v, sem):
        wid = lax.axis_index("s") * 2 + lax.axis_index("c")
        pltpu.sync_copy(idx.at[wid], idx_v)
        cps = [
            pltpu.async_copy(
                table.at[idx_v.at[j]], rows_v.at[pl.ds(j * _ECH, _ECH)], sem
            )
            for j in range(_EK)
        ]
        for cp in cps:
            cp.wait()
        pltpu.sync_copy(rows_v, out.at[pl.ds(wid * _EPW, _EPW)])

    return gk


def _make_scatter():
    rpt = _NPAD // 16

    @functools.partial(
        pl.kernel,
        out_type=jax.ShapeDtypeStruct((2, _NPAD, _H), jnp.float32),
        mesh=_sc_mesh(),
        compiler_params=_SC_PARAMS,
        scratch_types=[
            pltpu.VMEM((_EK, _ECH), jnp.int32),
            pltpu.VMEM((_EPW, _H), jnp.float32),
            pltpu.VMEM_SHARED((_NPAD, _H), jnp.float32),
            pltpu.SemaphoreType.DMA,
        ],
    )
    def sk(msg, idx, zeros, out, idx_v, rows_v, accum, sem):
        c = lax.axis_index("c")
        s = lax.axis_index("s")
        wid = s * 2 + c
        pltpu.sync_copy(zeros.at[pl.ds(s * rpt, rpt)], accum.at[pl.ds(s * rpt, rpt)])
        pltpu.sync_copy(idx.at[wid], idx_v)
        pltpu.sync_copy(msg.at[pl.ds(wid * _EPW, _EPW)], rows_v)
        plsc.subcore_barrier()
        for j in range(_EK):
            pltpu.sync_copy(
                rows_v.at[pl.ds(j * _ECH, _ECH)], accum.at[idx_v.at[j]], add=True
            )
        plsc.subcore_barrier()
        pltpu.sync_copy(accum.at[pl.ds(s * rpt, rpt)], out.at[c, pl.ds(s * rpt, rpt)])

    return sk


def _make_pool():
    rpt = _GP // 16

    @functools.partial(
        pl.kernel,
        out_type=jax.ShapeDtypeStruct((2, _GP, _HX), jnp.float32),
        mesh=_sc_mesh(),
        compiler_params=_SC_PARAMS,
        scratch_types=[
            pltpu.VMEM((_NK, _NCH), jnp.int32),
            pltpu.VMEM((_NPW, _HX), jnp.float32),
            pltpu.VMEM_SHARED((_GP, _HX), jnp.float32),
            pltpu.SemaphoreType.DMA,
        ],
    )
    def pk(hx, idx, zeros, out, idx_v, rows_v, accum, sem):
        c = lax.axis_index("c")
        s = lax.axis_index("s")
        wid = s * 2 + c
        pltpu.sync_copy(zeros.at[pl.ds(s * rpt, rpt)], accum.at[pl.ds(s * rpt, rpt)])
        pltpu.sync_copy(idx.at[wid], idx_v)
        pltpu.sync_copy(hx.at[pl.ds(wid * _NPW, _NPW)], rows_v)
        plsc.subcore_barrier()
        for j in range(_NK):
            pltpu.sync_copy(
                rows_v.at[pl.ds(j * _NCH, _NCH)], accum.at[idx_v.at[j]], add=True
            )
        plsc.subcore_barrier()
        pltpu.sync_copy(accum.at[pl.ds(s * rpt, rpt)], out.at[c, pl.ds(s * rpt, rpt)])

    return pk


def _edge_call(xs, ea, wall, sel, fold, bm, ic):
    te = 1024
    grid = _EP // te

    def body(xs_ref, ea_ref, w_ref, sel_ref, f_ref, b_ref, o_ref):
        xv = xs_ref[...]
        ev = ea_ref[...]
        s = jnp.dot(ev, sel_ref[...], preferred_element_type=jnp.float32)
        acc = None
        for q in range(4):
            y = jnp.dot(
                xv,
                w_ref[:, 128 * q : 128 * (q + 1)],
                preferred_element_type=jnp.float32,
            )
            z = s[:, 128 * q : 128 * (q + 1)] * y
            acc = z if acc is None else acc + z
        o_ref[...] = jnp.dot(
            acc, f_ref[...], preferred_element_type=jnp.float32
        ) + jnp.dot(xv, b_ref[...], preferred_element_type=jnp.float32)

    return pl.pallas_call(
        body,
        grid=(grid,),
        in_specs=[
            pl.BlockSpec((te, ic), lambda i: (i, 0)),
            pl.BlockSpec((te, _ED), lambda i: (i, 0)),
            pl.BlockSpec((ic, _ED * _H), lambda i: (0, 0)),
            pl.BlockSpec((_ED, _ED * _H), lambda i: (0, 0)),
            pl.BlockSpec((4 * _H, _H), lambda i: (0, 0)),
            pl.BlockSpec((ic, _H), lambda i: (0, 0)),
        ],
        out_specs=pl.BlockSpec((te, _H), lambda i: (i, 0)),
        out_shape=jax.ShapeDtypeStruct((_EP, _H), jnp.float32),
    )(xs, ea, wall, sel, fold, bm)


def _node_call(p, xin, root, bias, ic):
    tn = 2000
    grid = _N // tn

    def body(p_ref, x_ref, r_ref, b_ref, o_ref):
        acc = (
            p_ref[0]
            + p_ref[1]
            + jnp.dot(x_ref[...], r_ref[...], preferred_element_type=jnp.float32)
            + b_ref[...]
        )
        o_ref[...] = jnp.maximum(acc, 0.0)

    return pl.pallas_call(
        body,
        grid=(grid,),
        in_specs=[
            pl.BlockSpec((2, tn, _H), lambda i: (0, i, 0)),
            pl.BlockSpec((tn, ic), lambda i: (i, 0)),
            pl.BlockSpec((ic, _H), lambda i: (0, 0)),
            pl.BlockSpec((1, _H), lambda i: (0, 0)),
        ],
        out_specs=pl.BlockSpec((tn, _H), lambda i: (i, 0)),
        out_shape=jax.ShapeDtypeStruct((_N, _H), jnp.float32),
    )(p, xin, root, bias.reshape(1, _H))


def _node3_call(p, xin, root, bias):
    tn = 2000
    grid = _N // tn

    def body(p_ref, x_ref, r_ref, b_ref, emb_ref, hx_ref):
        emb = (
            p_ref[0]
            + p_ref[1]
            + jnp.dot(x_ref[...], r_ref[...], preferred_element_type=jnp.float32)
            + b_ref[...]
        )
        emb_ref[...] = emb
        hr = jnp.maximum(emb, 0.0)
        hx_ref[...] = jnp.concatenate(
            [hr, jnp.ones((tn, _HX - _H), jnp.float32)], axis=1
        )

    return pl.pallas_call(
        body,
        grid=(grid,),
        in_specs=[
            pl.BlockSpec((2, tn, _H), lambda i: (0, i, 0)),
            pl.BlockSpec((tn, _H), lambda i: (i, 0)),
            pl.BlockSpec((_H, _H), lambda i: (0, 0)),
            pl.BlockSpec((1, _H), lambda i: (0, 0)),
        ],
        out_specs=[
            pl.BlockSpec((tn, _H), lambda i: (i, 0)),
            pl.BlockSpec((tn, _HX), lambda i: (i, 0)),
        ],
        out_shape=[
            jax.ShapeDtypeStruct((_N, _H), jnp.float32),
            jax.ShapeDtypeStruct((_N, _HX), jnp.float32),
        ],
    )(p, xin, root, bias.reshape(1, _H))


def _head_call(pp, P1, pb1, P2, pb2):
    def body(p_ref, w1_ref, b1_ref, w2_ref, b2_ref, o_ref):
        s = p_ref[0] + p_ref[1]
        tot = s[:, : _H]
        cnt = s[:, _H : _H + 1]
        pooled = tot / jnp.maximum(cnt, 1.0)
        t = (
            jnp.dot(pooled, w1_ref[...], preferred_element_type=jnp.float32)
            + b1_ref[...]
        )
        o_ref[...] = (
            jnp.dot(t, w2_ref[...], preferred_element_type=jnp.float32) + b2_ref[...]
        )

    return pl.pallas_call(
        body,
        grid=(1,),
        in_specs=[
            pl.BlockSpec((2, _GP, _HX), lambda i: (0, 0, 0)),
            pl.BlockSpec((_H, _H), lambda i: (0, 0)),
            pl.BlockSpec((1, _H), lambda i: (0, 0)),
            pl.BlockSpec((_H, _OUT), lambda i: (0, 0)),
            pl.BlockSpec((1, _OUT), lambda i: (0, 0)),
        ],
        out_specs=pl.BlockSpec((_GP, _OUT), lambda i: (0, 0)),
        out_shape=jax.ShapeDtypeStruct((_GP, _OUT), jnp.float32),
    )(pp, P1, pb1.reshape(1, _H), P2, pb2.reshape(1, _OUT))


def kernel(x, edge_index, edge_attr, batch,
           We1, be1, root1, bias1,
           We2, be2, root2, bias2,
           root3, bias3,
           P1, pb1, P2, pb2):
    f32 = jnp.float32
    src = edge_index[0].astype(jnp.int32)
    dst = edge_index[1].astype(jnp.int32)
    srcp = jnp.concatenate([src, jnp.zeros((_EP - _E,), jnp.int32)]).reshape(
        _NW, _EK, _ECH
    )
    dstp = jnp.concatenate([dst, jnp.full((_EP - _E,), _N, jnp.int32)]).reshape(
        _NW, _EK, _ECH
    )
    bat = jnp.concatenate(
        [batch.astype(jnp.int32), jnp.full((_NP - _N,), _G, jnp.int32)]
    ).reshape(_NW, _NK, _NCH)
    eap = jnp.concatenate([edge_attr, jnp.zeros((_EP - _E, _ED), f32)])
    zn = jnp.zeros((_NPAD, _H), f32)
    zg = jnp.zeros((_GP, _HX), f32)
    wall1 = We1.reshape(_ED, _IN, _H).transpose(1, 0, 2).reshape(_IN, _ED * _H)
    bm1 = be1.reshape(_IN, _H)
    wall2 = We2.reshape(_ED, _H, _H).transpose(1, 0, 2).reshape(_H, _ED * _H)
    bm2 = be2.reshape(_H, _H)
    sel = jnp.repeat(jnp.eye(_ED, dtype=f32), _H, axis=1)
    fold = jnp.tile(jnp.eye(_H, dtype=f32), (4, 1))

    g64 = _make_gather(_IN)
    g32 = _make_gather(_H)
    sk = _make_scatter()
    pk = _make_pool()

    xs1 = g64(x, srcp)
    m1 = _edge_call(xs1, eap, wall1, sel, fold, bm1, _IN)
    p1 = sk(m1, dstp, zn)
    h1 = _node_call(p1, x, root1, bias1, _IN)

    xs2 = g32(h1, srcp)
    m2 = _edge_call(xs2, eap, wall2, sel, fold, bm2, _H)
    p2 = sk(m2, dstp, zn)
    h2 = _node_call(p2, h1, root2, bias2, _H)

    xs3 = g32(h2, srcp)
    m3 = _edge_call(xs3, eap, wall2, sel, fold, bm2, _H)
    p3 = sk(m3, dstp, zn)
    emb, hext = _node3_call(p3, h2, root3, bias3)

    hxp = jnp.concatenate([hext, jnp.zeros((_NP - _N, _HX), f32)])
    pp = pk(hxp, bat, zg)
    out = _head_call(pp, P1, pb1, P2, pb2)[: _G]
    return (emb, out)


# speedup vs baseline: 1.8145x; 1.0802x over previous
import functools

import jax
import jax.numpy as jnp
from jax import lax
from jax.experimental import pallas as pl
from jax.experimental.pallas import tpu as pltpu
from jax.experimental.pallas import tpu_sc as plsc

_N, _E, _IN, _H, _OUT, _ED, _G = 10000, 40000, 64, 32, 10, 16, 500

_NW = 32
_EP = _E
_EPW = 1250
_ECH = 125
_EK = _EPW // _ECH
_NPAD = _N
_NP = 10016
_NPW = 313
_NK = 3
_GP = 512
_HX = _H + 16


def _sc_mesh():
    return plsc.VectorSubcoreMesh(
        core_axis_name="c", subcore_axis_name="s", num_cores=2, num_subcores=16
    )


_SC_PARAMS = pltpu.CompilerParams(use_tc_tiling_on_sc=False)


def _make_gather(d):
    @functools.partial(
        pl.kernel,
        out_type=jax.ShapeDtypeStruct((_EP, d), jnp.float32),
        mesh=_sc_mesh(),
        compiler_params=_SC_PARAMS,
        scratch_types=[
            pltpu.VMEM((_EK, _ECH), jnp.int32),
            pltpu.VMEM((_EPW, d), jnp.float32),
            pltpu.SemaphoreType.DMA,
        ],
    )
    def gk(table, idx, out, idx_v, rows_v, sem):
        wid = lax.axis_index("s") * 2 + lax.axis_index("c")
        pltpu.sync_copy(idx.at[wid], idx_v)
        cps = [
            pltpu.async_copy(
                table.at[idx_v.at[j]], rows_v.at[pl.ds(j * _ECH, _ECH)], sem
            )
            for j in range(_EK)
        ]
        for cp in cps:
            cp.wait()
        pltpu.sync_copy(rows_v, out.at[pl.ds(wid * _EPW, _EPW)])

    return gk


def _make_scatter():
    rpt = _NPAD // 16

    @functools.partial(
        pl.kernel,
        out_type=jax.ShapeDtypeStruct((2, _NPAD, _H), jnp.float32),
        mesh=_sc_mesh(),
        compiler_params=_SC_PARAMS,
        scratch_types=[
            pltpu.VMEM((_EK, _ECH), jnp.int32),
            pltpu.VMEM((_EPW, _H), jnp.float32),
            pltpu.VMEM_SHARED((_NPAD, _H), jnp.float32),
            pltpu.SemaphoreType.DMA,
        ],
    )
    def sk(msg, idx, zeros, out, idx_v, rows_v, accum, sem):
        c = lax.axis_index("c")
        s = lax.axis_index("s")
        wid = s * 2 + c
        pltpu.sync_copy(zeros.at[pl.ds(s * rpt, rpt)], accum.at[pl.ds(s * rpt, rpt)])
        pltpu.sync_copy(idx.at[wid], idx_v)
        pltpu.sync_copy(msg.at[pl.ds(wid * _EPW, _EPW)], rows_v)
        plsc.subcore_barrier()
        for j in range(_EK):
            pltpu.sync_copy(
                rows_v.at[pl.ds(j * _ECH, _ECH)], accum.at[idx_v.at[j]], add=True
            )
        plsc.subcore_barrier()
        pltpu.sync_copy(accum.at[pl.ds(s * rpt, rpt)], out.at[c, pl.ds(s * rpt, rpt)])

    return sk


def _make_pool():
    rpt = _GP // 16

    @functools.partial(
        pl.kernel,
        out_type=jax.ShapeDtypeStruct((2, _GP, _HX), jnp.float32),
        mesh=_sc_mesh(),
        compiler_params=_SC_PARAMS,
        scratch_types=[
            pltpu.VMEM((_NK, 128), jnp.int32),
            pltpu.VMEM((_NK * 128, _HX), jnp.float32),
            pltpu.VMEM_SHARED((_GP, _HX), jnp.float32),
            pltpu.SemaphoreType.DMA,
        ],
    )
    def pk(hx, idx, zeros, out, idx_v, rows_v, accum, sem):
        c = lax.axis_index("c")
        s = lax.axis_index("s")
        wid = s * 2 + c
        pltpu.sync_copy(zeros.at[pl.ds(s * rpt, rpt)], accum.at[pl.ds(s * rpt, rpt)])
        pltpu.sync_copy(idx.at[wid], idx_v)
        pltpu.sync_copy(hx.at[pl.ds(wid * _NPW, _NPW)], rows_v.at[pl.ds(0, _NPW)])
        plsc.subcore_barrier()
        for j in range(_NK):
            pltpu.sync_copy(
                rows_v.at[pl.ds(j * 128, 128)], accum.at[idx_v.at[j]], add=True
            )
        plsc.subcore_barrier()
        pltpu.sync_copy(accum.at[pl.ds(s * rpt, rpt)], out.at[c, pl.ds(s * rpt, rpt)])

    return pk


def _edge_call(xs, ea, wall, sel, fold, bm, ic):
    te = 1000
    grid = _EP // te

    def body(xs_ref, ea_ref, w_ref, sel_ref, f_ref, b_ref, o_ref):
        xv = xs_ref[...]
        ev = ea_ref[...]
        s = jnp.dot(ev, sel_ref[...], preferred_element_type=jnp.float32)
        acc = None
        for q in range(4):
            y = jnp.dot(
                xv,
                w_ref[:, 128 * q : 128 * (q + 1)],
                preferred_element_type=jnp.float32,
            )
            z = s[:, 128 * q : 128 * (q + 1)] * y
            acc = z if acc is None else acc + z
        o_ref[...] = jnp.dot(
            acc, f_ref[...], preferred_element_type=jnp.float32
        ) + jnp.dot(xv, b_ref[...], preferred_element_type=jnp.float32)

    return pl.pallas_call(
        body,
        grid=(grid,),
        in_specs=[
            pl.BlockSpec((te, ic), lambda i: (i, 0)),
            pl.BlockSpec((te, _ED), lambda i: (i, 0)),
            pl.BlockSpec((ic, _ED * _H), lambda i: (0, 0)),
            pl.BlockSpec((_ED, _ED * _H), lambda i: (0, 0)),
            pl.BlockSpec((4 * _H, _H), lambda i: (0, 0)),
            pl.BlockSpec((ic, _H), lambda i: (0, 0)),
        ],
        out_specs=pl.BlockSpec((te, _H), lambda i: (i, 0)),
        out_shape=jax.ShapeDtypeStruct((_EP, _H), jnp.float32),
    )(xs, ea, wall, sel, fold, bm)


def _node_call(p, xin, root, bias, ic):
    tn = 2000
    grid = _N // tn

    def body(p_ref, x_ref, r_ref, b_ref, o_ref):
        acc = (
            p_ref[0]
            + p_ref[1]
            + jnp.dot(x_ref[...], r_ref[...], preferred_element_type=jnp.float32)
            + b_ref[...]
        )
        o_ref[...] = jnp.maximum(acc, 0.0)

    return pl.pallas_call(
        body,
        grid=(grid,),
        in_specs=[
            pl.BlockSpec((2, tn, _H), lambda i: (0, i, 0)),
            pl.BlockSpec((tn, ic), lambda i: (i, 0)),
            pl.BlockSpec((ic, _H), lambda i: (0, 0)),
            pl.BlockSpec((1, _H), lambda i: (0, 0)),
        ],
        out_specs=pl.BlockSpec((tn, _H), lambda i: (i, 0)),
        out_shape=jax.ShapeDtypeStruct((_N, _H), jnp.float32),
    )(p, xin, root, bias.reshape(1, _H))


def _node3_call(p, xin, root, bias):
    tn = 2000
    grid = _N // tn

    def body(p_ref, x_ref, r_ref, b_ref, emb_ref, hx_ref):
        emb = (
            p_ref[0]
            + p_ref[1]
            + jnp.dot(x_ref[...], r_ref[...], preferred_element_type=jnp.float32)
            + b_ref[...]
        )
        emb_ref[...] = emb
        hr = jnp.maximum(emb, 0.0)
        hx_ref[...] = jnp.concatenate(
            [hr, jnp.ones((tn, _HX - _H), jnp.float32)], axis=1
        )

    return pl.pallas_call(
        body,
        grid=(grid,),
        in_specs=[
            pl.BlockSpec((2, tn, _H), lambda i: (0, i, 0)),
            pl.BlockSpec((tn, _H), lambda i: (i, 0)),
            pl.BlockSpec((_H, _H), lambda i: (0, 0)),
            pl.BlockSpec((1, _H), lambda i: (0, 0)),
        ],
        out_specs=[
            pl.BlockSpec((tn, _H), lambda i: (i, 0)),
            pl.BlockSpec((tn, _HX), lambda i: (i, 0)),
        ],
        out_shape=[
            jax.ShapeDtypeStruct((_N, _H), jnp.float32),
            jax.ShapeDtypeStruct((_N, _HX), jnp.float32),
        ],
    )(p, xin, root, bias.reshape(1, _H))


def _head_call(pp, P1, pb1, P2, pb2):
    def body(p_ref, w1_ref, b1_ref, w2_ref, b2_ref, o_ref):
        s = p_ref[0] + p_ref[1]
        tot = s[:, : _H]
        cnt = s[:, _H : _H + 1]
        pooled = tot / jnp.maximum(cnt, 1.0)
        t = (
            jnp.dot(pooled, w1_ref[...], preferred_element_type=jnp.float32)
            + b1_ref[...]
        )
        o_ref[...] = (
            jnp.dot(t, w2_ref[...], preferred_element_type=jnp.float32) + b2_ref[...]
        )

    return pl.pallas_call(
        body,
        grid=(1,),
        in_specs=[
            pl.BlockSpec((2, _GP, _HX), lambda i: (0, 0, 0)),
            pl.BlockSpec((_H, _H), lambda i: (0, 0)),
            pl.BlockSpec((1, _H), lambda i: (0, 0)),
            pl.BlockSpec((_H, _OUT), lambda i: (0, 0)),
            pl.BlockSpec((1, _OUT), lambda i: (0, 0)),
        ],
        out_specs=pl.BlockSpec((_GP, _OUT), lambda i: (0, 0)),
        out_shape=jax.ShapeDtypeStruct((_GP, _OUT), jnp.float32),
    )(pp, P1, pb1.reshape(1, _H), P2, pb2.reshape(1, _OUT))


def kernel(x, edge_index, edge_attr, batch,
           We1, be1, root1, bias1,
           We2, be2, root2, bias2,
           root3, bias3,
           P1, pb1, P2, pb2):
    f32 = jnp.float32
    src = edge_index[0].astype(jnp.int32)
    dst = edge_index[1].astype(jnp.int32)
    srcp = src.reshape(_NW, _EK, _ECH)
    dstp = dst.reshape(_NW, _EK, _ECH)
    bat = jnp.pad(
        jnp.pad(batch.astype(jnp.int32), (0, _NP - _N), constant_values=_G)
        .reshape(_NW, _NPW),
        ((0, 0), (0, _NK * 128 - _NPW)),
        constant_values=_G,
    ).reshape(_NW, _NK, 128)
    eap = edge_attr
    zn = jnp.zeros((_NPAD, _H), f32)
    zg = jnp.zeros((_GP, _HX), f32)
    wall1 = We1.reshape(_ED, _IN, _H).transpose(1, 0, 2).reshape(_IN, _ED * _H)
    bm1 = be1.reshape(_IN, _H)
    wall2 = We2.reshape(_ED, _H, _H).transpose(1, 0, 2).reshape(_H, _ED * _H)
    bm2 = be2.reshape(_H, _H)
    sel = jnp.repeat(jnp.eye(_ED, dtype=f32), _H, axis=1)
    fold = jnp.tile(jnp.eye(_H, dtype=f32), (4, 1))

    g64 = _make_gather(_IN)
    g32 = _make_gather(_H)
    sk = _make_scatter()
    pk = _make_pool()

    xs1 = g64(x, srcp)
    m1 = _edge_call(xs1, eap, wall1, sel, fold, bm1, _IN)
    p1 = sk(m1, dstp, zn)
    h1 = _node_call(p1, x, root1, bias1, _IN)

    xs2 = g32(h1, srcp)
    m2 = _edge_call(xs2, eap, wall2, sel, fold, bm2, _H)
    p2 = sk(m2, dstp, zn)
    h2 = _node_call(p2, h1, root2, bias2, _H)

    xs3 = g32(h2, srcp)
    m3 = _edge_call(xs3, eap, wall2, sel, fold, bm2, _H)
    p3 = sk(m3, dstp, zn)
    emb, hext = _node3_call(p3, h2, root3, bias3)

    hxp = jnp.pad(hext, ((0, _NP - _N), (0, 0)))
    pp = pk(hxp, bat, zg)
    out = _head_call(pp, P1, pb1, P2, pb2)[: _G]
    return (emb, out)
